# baseline (speedup 1.0000x reference)
import functools

import jax
import jax.numpy as jnp
from jax import lax
from jax.experimental import pallas as pl
from jax.experimental.pallas import tpu as pltpu
from jax.experimental.pallas import tpu_sc as plsc

B = 16384
NUM = 13
F = 26
V = 100000
D = 32
H = 256
C = 7

_NC, _NS = 2, 16
_NW = _NC * _NS
_RPW = B * F // _NW
_CHUNK = 128
_NCHUNK = _RPW // _CHUNK
_KF = 8
_NOUT = _NCHUNK // _KF

BLK = 512
NB = B // BLK
TBLK = 2048


def _sc_gather(table, idx3):
    mesh = plsc.VectorSubcoreMesh(core_axis_name="c", subcore_axis_name="s")
    cpf = B // _CHUNK

    @functools.partial(
        pl.kernel,
        mesh=mesh,
        out_type=jax.ShapeDtypeStruct((F * B, D), jnp.float32),
        scratch_types=[
            pltpu.VMEM((_NCHUNK, _CHUNK), jnp.int32),
            pltpu.VMEM((_KF * _CHUNK, D), jnp.float32),
            pltpu.SemaphoreType.DMA,
        ],
        compiler_params=pltpu.CompilerParams(use_tc_tiling_on_sc=False),
    )
    def gk(table_hbm, idx_hbm, out_hbm, idx_v, rows_v, sem):
        wid = lax.axis_index("s") * _NC + lax.axis_index("c")
        base = wid * _RPW
        cbase = wid * _NCHUNK
        pltpu.sync_copy(idx_hbm.at[wid], idx_v)

        def outer(t, carry):
            cps = []
            for k in range(_KF):
                f = (cbase + t * _KF + k) // cpf
                cps.append(pltpu.async_copy(
                    table_hbm.at[f].at[idx_v.at[t * _KF + k]],
                    rows_v.at[pl.ds(k * _CHUNK, _CHUNK)],
                    sem))
            for cp in cps:
                cp.wait()
            pltpu.sync_copy(
                rows_v,
                out_hbm.at[pl.ds(base + t * (_KF * _CHUNK), _KF * _CHUNK)])
            return carry

        lax.fori_loop(0, _NOUT, outer, 0)

    return gk(table, idx3)


def _mlp_body(xnum_ref, embc_ref, w0n_ref, w0e_ref, b0_ref, g0_ref, be0_ref,
              w1_ref, b1_ref, g1_ref, be1_ref,
              w2_ref, b2_ref, g2_ref, be2_ref,
              wout_ref, bout_ref, out_ref, zbuf, acc):
    t = pl.program_id(0)

    @pl.when(t < NB)
    def _layer0():
        @pl.when(t == 0)
        def _init():
            acc[...] = jnp.zeros_like(acc[...])

        z = (jnp.dot(xnum_ref[...], w0n_ref[...],
                     preferred_element_type=jnp.float32)
             + jnp.dot(embc_ref[...], w0e_ref[...],
                       preferred_element_type=jnp.float32)
             + b0_ref[...])
        zbuf[pl.ds(t * BLK, BLK), :] = z
        acc[0:1, :] += jnp.sum(z, axis=0, keepdims=True)
        acc[1:2, :] += jnp.sum(z * z, axis=0, keepdims=True)

    def _affine(g_ref, be_ref):
        mean = acc[0:1, :] * (1.0 / B)
        var = acc[1:2, :] * (1.0 / B) - mean * mean
        scale = g_ref[...] * lax.rsqrt(var + 1e-5)
        shift = be_ref[...] - mean * scale
        return scale, shift

    def _mid(w_ref, b_ref, g_ref, be_ref):
        scale, shift = _affine(g_ref, be_ref)

        def body(j, s):
            zs = zbuf[pl.ds(j * TBLK, TBLK), :]
            h = jnp.maximum(zs * scale + shift, 0.0)
            z = jnp.dot(h, w_ref[...],
                        preferred_element_type=jnp.float32) + b_ref[...]
            zbuf[pl.ds(j * TBLK, TBLK), :] = z
            return (s[0] + jnp.sum(z, axis=0, keepdims=True),
                    s[1] + jnp.sum(z * z, axis=0, keepdims=True))

        s0 = (jnp.zeros((1, H), jnp.float32), jnp.zeros((1, H), jnp.float32))
        s = lax.fori_loop(0, B // TBLK, body, s0)
        acc[0:1, :] = s[0]
        acc[1:2, :] = s[1]

    @pl.when(t == NB)
    def _layer1():
        _mid(w1_ref, b1_ref, g0_ref, be0_ref)

    @pl.when(t == NB + 1)
    def _layer2():
        _mid(w2_ref, b2_ref, g1_ref, be1_ref)

    @pl.when(t == NB + 2)
    def _head():
        scale, shift = _affine(g2_ref, be2_ref)

        def body(j, carry):
            zs = zbuf[pl.ds(j * TBLK, TBLK), :]
            h = jnp.maximum(zs * scale + shift, 0.0)
            out_ref[pl.ds(j * TBLK, TBLK), :] = (
                jnp.dot(h, wout_ref[...],
                        preferred_element_type=jnp.float32) + bout_ref[...])
            return carry

        lax.fori_loop(0, B // TBLK, body, 0)


def _tc_mlp(xnum, embc, w0n, w0e, b0, g0, be0,
            W1, b1, g1, be1, W2, b2, g2, be2, Wout, bout):
    def full(shape):
        return pl.BlockSpec(shape, lambda t: (0, 0))

    def inb(t):
        return (jnp.minimum(t, NB - 1), 0)

    return pl.pallas_call(
        _mlp_body,
        grid=(NB + 3,),
        in_specs=[
            pl.BlockSpec((BLK, NUM), inb),
            pl.BlockSpec((BLK, F * D), inb),
            full((NUM, H)), full((F * D, H)),
            full((1, H)), full((1, H)), full((1, H)),
            full((H, H)), full((1, H)), full((1, H)), full((1, H)),
            full((H, H)), full((1, H)), full((1, H)), full((1, H)),
            full((H, C)), full((1, C)),
        ],
        out_specs=pl.BlockSpec((B, C), lambda t: (0, 0)),
        out_shape=jax.ShapeDtypeStruct((B, C), jnp.float32),
        scratch_shapes=[
            pltpu.VMEM((B, H), jnp.float32),
            pltpu.VMEM((2, H), jnp.float32),
        ],
        compiler_params=pltpu.CompilerParams(
            dimension_semantics=("arbitrary",)),
    )(xnum, embc, w0n, w0e, b0, g0, be0,
      W1, b1, g1, be1, W2, b2, g2, be2, Wout, bout)


def kernel(x, emb, W0, b0, g0, be0, W1, b1, g1, be1, W2, b2, g2, be2,
           Wout, bout):
    xnum = x[:, :NUM]
    xcat = x[:, NUM:].astype(jnp.int32)
    idx3 = xcat.T.reshape(_NW, _NCHUNK, _CHUNK)
    rows = _sc_gather(emb, idx3)
    embc = rows.reshape(F, B, D).transpose(1, 0, 2).reshape(B, F * D)
    return _tc_mlp(
        xnum, embc, W0[:NUM], W0[NUM:],
        b0.reshape(1, H), g0.reshape(1, H), be0.reshape(1, H),
        W1, b1.reshape(1, H), g1.reshape(1, H), be1.reshape(1, H),
        W2, b2.reshape(1, H), g2.reshape(1, H), be2.reshape(1, H),
        Wout, bout.reshape(1, C))


# speedup vs baseline: 1.4614x; 1.4614x over previous
import functools

import jax
import jax.numpy as jnp
from jax import lax
from jax.experimental import pallas as pl
from jax.experimental.pallas import tpu as pltpu
from jax.experimental.pallas import tpu_sc as plsc

B = 16384
NUM = 13
F = 26
V = 100000
D = 32
H = 256
C = 7

TV = 4992
NJ = -(-V // TV)

_NC, _NS = 2, 16
_NW = _NC * _NS
_RPW = B * F // _NW
_CHUNK = 128
_NCHUNK = _RPW // _CHUNK
_KF = 4
_NOUT = _NCHUNK // _KF

BLK = 512
NB = B // BLK
TBLK = 2048


def _pad_body(in_ref, out_ref):
    x = in_ref[0]
    out_ref[0, :, :D] = jnp.transpose(x, (1, 0))


def _tc_pad(embT):
    return pl.pallas_call(
        _pad_body,
        grid=(F, NJ),
        in_specs=[pl.BlockSpec((1, D, TV), lambda f, j: (f, 0, j))],
        out_specs=pl.BlockSpec((1, TV, 128), lambda f, j: (f, j, 0)),
        out_shape=jax.ShapeDtypeStruct((F, V, 128), jnp.float32),
    )(embT)


def _sc_gather(table, idx3):
    mesh = plsc.VectorSubcoreMesh(core_axis_name="c", subcore_axis_name="s")

    @functools.partial(
        pl.kernel,
        mesh=mesh,
        out_type=jax.ShapeDtypeStruct((F * B, 128), jnp.float32),
        scratch_types=[
            pltpu.VMEM((_NCHUNK, _CHUNK), jnp.int32),
            pltpu.VMEM((_KF * _CHUNK, 128), jnp.float32),
            pltpu.SemaphoreType.DMA,
        ],
        compiler_params=pltpu.CompilerParams(use_tc_tiling_on_sc=True),
    )
    def gk(table_hbm, idx_hbm, out_hbm, idx_v, rows_v, sem):
        wid = lax.axis_index("s") * _NC + lax.axis_index("c")
        base = wid * _RPW
        pltpu.sync_copy(idx_hbm.at[wid], idx_v)

        def outer(t, carry):
            cps = []
            for k in range(_KF):
                cps.append(pltpu.async_copy(
                    table_hbm.at[idx_v.at[t * _KF + k]],
                    rows_v.at[pl.ds(k * _CHUNK, _CHUNK)],
                    sem))
            for cp in cps:
                cp.wait()
            pltpu.sync_copy(
                rows_v,
                out_hbm.at[pl.ds(base + t * (_KF * _CHUNK), _KF * _CHUNK)])
            return carry

        lax.fori_loop(0, _NOUT, outer, 0)

    return gk(table, idx3)


def _mlp_body(xnum_ref, embf_ref, w0n_ref, w0e_ref, b0_ref, g0_ref, be0_ref,
              w1_ref, b1_ref, g1_ref, be1_ref,
              w2_ref, b2_ref, g2_ref, be2_ref,
              wout_ref, bout_ref, out_ref, zbuf, acc, ebuf):
    t = pl.program_id(0)

    @pl.when(t < NB)
    def _layer0():
        @pl.when(t == 0)
        def _init():
            acc[...] = jnp.zeros_like(acc[...])

        for f in range(F):
            ebuf[:, f * D:(f + 1) * D] = embf_ref[f, :, :D]
        z = (jnp.dot(xnum_ref[...], w0n_ref[...],
                     preferred_element_type=jnp.float32)
             + jnp.dot(ebuf[...], w0e_ref[...],
                       preferred_element_type=jnp.float32)
             + b0_ref[...])
        zbuf[pl.ds(t * BLK, BLK), :] = z
        acc[0:1, :] += jnp.sum(z, axis=0, keepdims=True)
        acc[1:2, :] += jnp.sum(z * z, axis=0, keepdims=True)

    def _affine(g_ref, be_ref):
        mean = acc[0:1, :] * (1.0 / B)
        var = acc[1:2, :] * (1.0 / B) - mean * mean
        scale = g_ref[...] * lax.rsqrt(var + 1e-5)
        shift = be_ref[...] - mean * scale
        return scale, shift

    def _mid(w_ref, b_ref, g_ref, be_ref):
        scale, shift = _affine(g_ref, be_ref)

        def body(j, s):
            zs = zbuf[pl.ds(j * TBLK, TBLK), :]
            h = jnp.maximum(zs * scale + shift, 0.0)
            z = jnp.dot(h, w_ref[...],
                        preferred_element_type=jnp.float32) + b_ref[...]
            zbuf[pl.ds(j * TBLK, TBLK), :] = z
            return (s[0] + jnp.sum(z, axis=0, keepdims=True),
                    s[1] + jnp.sum(z * z, axis=0, keepdims=True))

        s0 = (jnp.zeros((1, H), jnp.float32), jnp.zeros((1, H), jnp.float32))
        s = lax.fori_loop(0, B // TBLK, body, s0)
        acc[0:1, :] = s[0]
        acc[1:2, :] = s[1]

    @pl.when(t == NB)
    def _layer1():
        _mid(w1_ref, b1_ref, g0_ref, be0_ref)

    @pl.when(t == NB + 1)
    def _layer2():
        _mid(w2_ref, b2_ref, g1_ref, be1_ref)

    @pl.when(t == NB + 2)
    def _head():
        scale, shift = _affine(g2_ref, be2_ref)

        def body(j, carry):
            zs = zbuf[pl.ds(j * TBLK, TBLK), :]
            h = jnp.maximum(zs * scale + shift, 0.0)
            out_ref[pl.ds(j * TBLK, TBLK), :] = (
                jnp.dot(h, wout_ref[...],
                        preferred_element_type=jnp.float32) + bout_ref[...])
            return carry

        lax.fori_loop(0, B // TBLK, body, 0)


def _tc_mlp(xnum, embf, w0n, w0e, b0, g0, be0,
            W1, b1, g1, be1, W2, b2, g2, be2, Wout, bout):
    def full(shape):
        return pl.BlockSpec(shape, lambda t: (0, 0))

    def inb(t):
        return (jnp.minimum(t, NB - 1), 0)

    return pl.pallas_call(
        _mlp_body,
        grid=(NB + 3,),
        in_specs=[
            pl.BlockSpec((BLK, NUM), inb),
            pl.BlockSpec((F, BLK, 128),
                         lambda t: (0, jnp.minimum(t, NB - 1), 0)),
            full((NUM, H)), full((F * D, H)),
            full((1, H)), full((1, H)), full((1, H)),
            full((H, H)), full((1, H)), full((1, H)), full((1, H)),
            full((H, H)), full((1, H)), full((1, H)), full((1, H)),
            full((H, C)), full((1, C)),
        ],
        out_specs=pl.BlockSpec((B, C), lambda t: (0, 0)),
        out_shape=jax.ShapeDtypeStruct((B, C), jnp.float32),
        scratch_shapes=[
            pltpu.VMEM((B, H), jnp.float32),
            pltpu.VMEM((2, H), jnp.float32),
            pltpu.VMEM((BLK, F * D), jnp.float32),
        ],
        compiler_params=pltpu.CompilerParams(
            dimension_semantics=("arbitrary",)),
    )(xnum, embf, w0n, w0e, b0, g0, be0,
      W1, b1, g1, be1, W2, b2, g2, be2, Wout, bout)


def kernel(x, emb, W0, b0, g0, be0, W1, b1, g1, be1, W2, b2, g2, be2,
           Wout, bout):
    xnum = x[:, :NUM]
    xcat = x[:, NUM:].astype(jnp.int32)
    idx_fm = xcat.T + (jnp.arange(F, dtype=jnp.int32) * V)[:, None]
    idx3 = idx_fm.reshape(_NW, _NCHUNK, _CHUNK)
    table = _tc_pad(emb.transpose(0, 2, 1)).reshape(F * V, 128)
    rows = _sc_gather(table, idx3)
    embf = rows.reshape(F, B, 128)
    return _tc_mlp(
        xnum, embf, W0[:NUM], W0[NUM:],
        b0.reshape(1, H), g0.reshape(1, H), be0.reshape(1, H),
        W1, b1.reshape(1, H), g1.reshape(1, H), be1.reshape(1, H),
        W2, b2.reshape(1, H), g2.reshape(1, H), be2.reshape(1, H),
        Wout, bout.reshape(1, C))


# speedup vs baseline: 1.8063x; 1.2360x over previous
import functools

import jax
import jax.numpy as jnp
from jax import lax
from jax.experimental import pallas as pl
from jax.experimental.pallas import tpu as pltpu
from jax.experimental.pallas import tpu_sc as plsc

B = 16384
NUM = 13
F = 26
V = 100000
D = 32
H = 256
C = 7

VP = 100352
TV = VP // 8
NJ = 8

_NC, _NS = 2, 16
_NW = _NC * _NS
_RPW = B * F // _NW
_CHUNK = 128
_NCHUNK = _RPW // _CHUNK
_KF = 4
_NOUT = _NCHUNK // _KF

BLK = 512
NB = B // BLK
TBLK = 2048


def _pad_body(in_ref, e_ref, out_ref):
    x = in_ref[0]
    out_ref[0] = lax.dot_general(
        x, e_ref[...], (((0,), (0,)), ((), ())),
        preferred_element_type=jnp.float32)


def _tc_pad(embT):
    eye = jnp.eye(D, 128, dtype=jnp.float32)
    return pl.pallas_call(
        _pad_body,
        grid=(F, NJ),
        in_specs=[
            pl.BlockSpec((1, D, TV), lambda f, j: (f, 0, j)),
            pl.BlockSpec((D, 128), lambda f, j: (0, 0)),
        ],
        out_specs=pl.BlockSpec((1, TV, 128), lambda f, j: (f, j, 0)),
        out_shape=jax.ShapeDtypeStruct((F, VP, 128), jnp.float32),
    )(embT, eye)


def _sc_gather(table, idx3):
    mesh = plsc.VectorSubcoreMesh(core_axis_name="c", subcore_axis_name="s")

    @functools.partial(
        pl.kernel,
        mesh=mesh,
        out_type=jax.ShapeDtypeStruct((F * B, 128), jnp.float32),
        scratch_types=[
            pltpu.VMEM((_NCHUNK, _CHUNK), jnp.int32),
            pltpu.VMEM((_KF * _CHUNK, 128), jnp.float32),
            pltpu.SemaphoreType.DMA,
        ],
        compiler_params=pltpu.CompilerParams(use_tc_tiling_on_sc=True),
    )
    def gk(table_hbm, idx_hbm, out_hbm, idx_v, rows_v, sem):
        wid = lax.axis_index("s") * _NC + lax.axis_index("c")
        base = wid * _RPW
        pltpu.sync_copy(idx_hbm.at[wid], idx_v)

        def outer(t, carry):
            cps = []
            for k in range(_KF):
                cps.append(pltpu.async_copy(
                    table_hbm.at[idx_v.at[t * _KF + k]],
                    rows_v.at[pl.ds(k * _CHUNK, _CHUNK)],
                    sem))
            for cp in cps:
                cp.wait()
            pltpu.sync_copy(
                rows_v,
                out_hbm.at[pl.ds(base + t * (_KF * _CHUNK), _KF * _CHUNK)])
            return carry

        lax.fori_loop(0, _NOUT, outer, 0)

    return gk(table, idx3)


def _mlp_body(xnum_ref, embf_ref, w0n_ref, w0e_ref, b0_ref, g0_ref, be0_ref,
              w1_ref, b1_ref, g1_ref, be1_ref,
              w2_ref, b2_ref, g2_ref, be2_ref,
              wout_ref, bout_ref, out_ref, zbuf, acc, ebuf):
    t = pl.program_id(0)

    @pl.when(t < NB)
    def _layer0():
        @pl.when(t == 0)
        def _init():
            acc[...] = jnp.zeros_like(acc[...])

        for f in range(F):
            ebuf[:, f * D:(f + 1) * D] = embf_ref[f, :, :D]
        z = (jnp.dot(xnum_ref[...], w0n_ref[...],
                     preferred_element_type=jnp.float32)
             + jnp.dot(ebuf[...], w0e_ref[...],
                       preferred_element_type=jnp.float32)
             + b0_ref[...])
        zbuf[pl.ds(t * BLK, BLK), :] = z
        acc[0:1, :] += jnp.sum(z, axis=0, keepdims=True)
        acc[1:2, :] += jnp.sum(z * z, axis=0, keepdims=True)

    def _affine(g_ref, be_ref):
        mean = acc[0:1, :] * (1.0 / B)
        var = acc[1:2, :] * (1.0 / B) - mean * mean
        scale = g_ref[...] * lax.rsqrt(var + 1e-5)
        shift = be_ref[...] - mean * scale
        return scale, shift

    def _mid(w_ref, b_ref, g_ref, be_ref):
        scale, shift = _affine(g_ref, be_ref)

        def body(j, s):
            zs = zbuf[pl.ds(j * TBLK, TBLK), :]
            h = jnp.maximum(zs * scale + shift, 0.0)
            z = jnp.dot(h, w_ref[...],
                        preferred_element_type=jnp.float32) + b_ref[...]
            zbuf[pl.ds(j * TBLK, TBLK), :] = z
            return (s[0] + jnp.sum(z, axis=0, keepdims=True),
                    s[1] + jnp.sum(z * z, axis=0, keepdims=True))

        s0 = (jnp.zeros((1, H), jnp.float32), jnp.zeros((1, H), jnp.float32))
        s = lax.fori_loop(0, B // TBLK, body, s0)
        acc[0:1, :] = s[0]
        acc[1:2, :] = s[1]

    @pl.when(t == NB)
    def _layer1():
        _mid(w1_ref, b1_ref, g0_ref, be0_ref)

    @pl.when(t == NB + 1)
    def _layer2():
        _mid(w2_ref, b2_ref, g1_ref, be1_ref)

    @pl.when(t == NB + 2)
    def _head():
        scale, shift = _affine(g2_ref, be2_ref)

        def body(j, carry):
            zs = zbuf[pl.ds(j * TBLK, TBLK), :]
            h = jnp.maximum(zs * scale + shift, 0.0)
            out_ref[pl.ds(j * TBLK, TBLK), :] = (
                jnp.dot(h, wout_ref[...],
                        preferred_element_type=jnp.float32) + bout_ref[...])
            return carry

        lax.fori_loop(0, B // TBLK, body, 0)


def _tc_mlp(xnum, embf, w0n, w0e, b0, g0, be0,
            W1, b1, g1, be1, W2, b2, g2, be2, Wout, bout):
    def full(shape):
        return pl.BlockSpec(shape, lambda t: (0, 0))

    def inb(t):
        return (jnp.minimum(t, NB - 1), 0)

    return pl.pallas_call(
        _mlp_body,
        grid=(NB + 3,),
        in_specs=[
            pl.BlockSpec((BLK, NUM), inb),
            pl.BlockSpec((F, BLK, 128),
                         lambda t: (0, jnp.minimum(t, NB - 1), 0)),
            full((NUM, H)), full((F * D, H)),
            full((1, H)), full((1, H)), full((1, H)),
            full((H, H)), full((1, H)), full((1, H)), full((1, H)),
            full((H, H)), full((1, H)), full((1, H)), full((1, H)),
            full((H, C)), full((1, C)),
        ],
        out_specs=pl.BlockSpec((B, C), lambda t: (0, 0)),
        out_shape=jax.ShapeDtypeStruct((B, C), jnp.float32),
        scratch_shapes=[
            pltpu.VMEM((B, H), jnp.float32),
            pltpu.VMEM((2, H), jnp.float32),
            pltpu.VMEM((BLK, F * D), jnp.float32),
        ],
        compiler_params=pltpu.CompilerParams(
            dimension_semantics=("arbitrary",)),
    )(xnum, embf, w0n, w0e, b0, g0, be0,
      W1, b1, g1, be1, W2, b2, g2, be2, Wout, bout)


def kernel(x, emb, W0, b0, g0, be0, W1, b1, g1, be1, W2, b2, g2, be2,
           Wout, bout):
    xnum = x[:, :NUM]
    xcat = x[:, NUM:].astype(jnp.int32)
    idx_fm = xcat.T + (jnp.arange(F, dtype=jnp.int32) * VP)[:, None]
    idx3 = idx_fm.reshape(_NW, _NCHUNK, _CHUNK)
    table = _tc_pad(emb.transpose(0, 2, 1)).reshape(F * VP, 128)
    rows = _sc_gather(table, idx3)
    embf = rows.reshape(F, B, 128)
    return _tc_mlp(
        xnum, embf, W0[:NUM], W0[NUM:],
        b0.reshape(1, H), g0.reshape(1, H), be0.reshape(1, H),
        W1, b1.reshape(1, H), g1.reshape(1, H), be1.reshape(1, H),
        W2, b2.reshape(1, H), g2.reshape(1, H), be2.reshape(1, H),
        Wout, bout.reshape(1, C))


# speedup vs baseline: 1.8973x; 1.0504x over previous
import functools

import jax
import jax.numpy as jnp
from jax import lax
from jax.experimental import pallas as pl
from jax.experimental.pallas import tpu as pltpu
from jax.experimental.pallas import tpu_sc as plsc

B = 16384
NUM = 13
F = 26
V = 100000
D = 32
H = 256
C = 7

VP = 100352
TV = VP // 4
NJ = 4

_NC, _NS = 2, 16
_NW = _NC * _NS
_RPW = B * F // _NW
_CHUNK = 128
_NCHUNK = _RPW // _CHUNK
_KF = 4
_NOUT = _NCHUNK // _KF

BLK = 512
NB = B // BLK
TBLK = 2048


def _pad_body(in_ref, e_ref, out_ref):
    x = in_ref[0]
    out_ref[0] = lax.dot_general(
        x, e_ref[...], (((0,), (0,)), ((), ())),
        preferred_element_type=jnp.float32)


def _tc_pad(embT):
    eye = jnp.eye(D, 128, dtype=jnp.float32)
    return pl.pallas_call(
        _pad_body,
        grid=(F, NJ),
        in_specs=[
            pl.BlockSpec((1, D, TV), lambda f, j: (f, 0, j)),
            pl.BlockSpec((D, 128), lambda f, j: (0, 0)),
        ],
        out_specs=pl.BlockSpec((1, TV, 128), lambda f, j: (f, j, 0)),
        out_shape=jax.ShapeDtypeStruct((F, VP, 128), jnp.float32),
    )(embT, eye)


def _sc_gather(table, idx3):
    mesh = plsc.VectorSubcoreMesh(core_axis_name="c", subcore_axis_name="s")

    @functools.partial(
        pl.kernel,
        mesh=mesh,
        out_type=jax.ShapeDtypeStruct((F * B, 128), jnp.float32),
        scratch_types=[
            pltpu.VMEM((_NCHUNK, _CHUNK), jnp.int32),
            pltpu.VMEM((_KF * _CHUNK, 128), jnp.float32),
            pltpu.SemaphoreType.DMA,
        ],
        compiler_params=pltpu.CompilerParams(use_tc_tiling_on_sc=True),
    )
    def gk(table_hbm, idx_hbm, out_hbm, idx_v, rows_v, sem):
        wid = lax.axis_index("s") * _NC + lax.axis_index("c")
        base = wid * _RPW
        pltpu.sync_copy(idx_hbm.at[wid], idx_v)

        def outer(t, carry):
            cps = []
            for k in range(_KF):
                cps.append(pltpu.async_copy(
                    table_hbm.at[idx_v.at[t * _KF + k]],
                    rows_v.at[pl.ds(k * _CHUNK, _CHUNK)],
                    sem))
            for cp in cps:
                cp.wait()
            pltpu.sync_copy(
                rows_v,
                out_hbm.at[pl.ds(base + t * (_KF * _CHUNK), _KF * _CHUNK)])
            return carry

        lax.fori_loop(0, _NOUT, outer, 0)

    return gk(table, idx3)


def _mlp_body(xnum_ref, embf_ref, w0n_ref, w0e_ref, b0_ref, g0_ref, be0_ref,
              w1_ref, b1_ref, g1_ref, be1_ref,
              w2_ref, b2_ref, g2_ref, be2_ref,
              wout_ref, bout_ref, out_ref, zbuf, acc, ebuf):
    t = pl.program_id(0)

    @pl.when(t < NB)
    def _layer0():
        @pl.when(t == 0)
        def _init():
            acc[...] = jnp.zeros_like(acc[...])

        for f in range(F):
            ebuf[:, f * D:(f + 1) * D] = embf_ref[f, :, :D]
        z = (jnp.dot(xnum_ref[...], w0n_ref[...],
                     preferred_element_type=jnp.float32)
             + jnp.dot(ebuf[...], w0e_ref[...],
                       preferred_element_type=jnp.float32)
             + b0_ref[...])
        zbuf[pl.ds(t * BLK, BLK), :] = z
        acc[0:1, :] += jnp.sum(z, axis=0, keepdims=True)
        acc[1:2, :] += jnp.sum(z * z, axis=0, keepdims=True)

    def _affine(g_ref, be_ref):
        mean = acc[0:1, :] * (1.0 / B)
        var = acc[1:2, :] * (1.0 / B) - mean * mean
        scale = g_ref[...] * lax.rsqrt(var + 1e-5)
        shift = be_ref[...] - mean * scale
        return scale, shift

    def _mid(w_ref, b_ref, g_ref, be_ref):
        scale, shift = _affine(g_ref, be_ref)

        def body(j, s):
            zs = zbuf[pl.ds(j * TBLK, TBLK), :]
            h = jnp.maximum(zs * scale + shift, 0.0)
            z = jnp.dot(h, w_ref[...],
                        preferred_element_type=jnp.float32) + b_ref[...]
            zbuf[pl.ds(j * TBLK, TBLK), :] = z
            return (s[0] + jnp.sum(z, axis=0, keepdims=True),
                    s[1] + jnp.sum(z * z, axis=0, keepdims=True))

        s0 = (jnp.zeros((1, H), jnp.float32), jnp.zeros((1, H), jnp.float32))
        s = lax.fori_loop(0, B // TBLK, body, s0)
        acc[0:1, :] = s[0]
        acc[1:2, :] = s[1]

    @pl.when(t == NB)
    def _layer1():
        _mid(w1_ref, b1_ref, g0_ref, be0_ref)

    @pl.when(t == NB + 1)
    def _layer2():
        _mid(w2_ref, b2_ref, g1_ref, be1_ref)

    @pl.when(t == NB + 2)
    def _head():
        scale, shift = _affine(g2_ref, be2_ref)

        def body(j, carry):
            zs = zbuf[pl.ds(j * TBLK, TBLK), :]
            h = jnp.maximum(zs * scale + shift, 0.0)
            out_ref[pl.ds(j * TBLK, TBLK), :] = (
                jnp.dot(h, wout_ref[...],
                        preferred_element_type=jnp.float32) + bout_ref[...])
            return carry

        lax.fori_loop(0, B // TBLK, body, 0)


def _tc_mlp(xnum, embf, w0n, w0e, b0, g0, be0,
            W1, b1, g1, be1, W2, b2, g2, be2, Wout, bout):
    def full(shape):
        return pl.BlockSpec(shape, lambda t: (0, 0))

    def inb(t):
        return (jnp.minimum(t, NB - 1), 0)

    return pl.pallas_call(
        _mlp_body,
        grid=(NB + 3,),
        in_specs=[
            pl.BlockSpec((BLK, NUM), inb),
            pl.BlockSpec((F, BLK, 128),
                         lambda t: (0, jnp.minimum(t, NB - 1), 0)),
            full((NUM, H)), full((F * D, H)),
            full((1, H)), full((1, H)), full((1, H)),
            full((H, H)), full((1, H)), full((1, H)), full((1, H)),
            full((H, H)), full((1, H)), full((1, H)), full((1, H)),
            full((H, C)), full((1, C)),
        ],
        out_specs=pl.BlockSpec((B, C), lambda t: (0, 0)),
        out_shape=jax.ShapeDtypeStruct((B, C), jnp.float32),
        scratch_shapes=[
            pltpu.VMEM((B, H), jnp.float32),
            pltpu.VMEM((2, H), jnp.float32),
            pltpu.VMEM((BLK, F * D), jnp.float32),
        ],
        compiler_params=pltpu.CompilerParams(
            dimension_semantics=("arbitrary",)),
    )(xnum, embf, w0n, w0e, b0, g0, be0,
      W1, b1, g1, be1, W2, b2, g2, be2, Wout, bout)


def kernel(x, emb, W0, b0, g0, be0, W1, b1, g1, be1, W2, b2, g2, be2,
           Wout, bout):
    xnum = x[:, :NUM]
    xcat = x[:, NUM:].astype(jnp.int32)
    idx_fm = xcat.T + (jnp.arange(F, dtype=jnp.int32) * VP)[:, None]
    idx3 = idx_fm.reshape(_NW, _NCHUNK, _CHUNK)
    table = _tc_pad(emb.transpose(0, 2, 1)).reshape(F * VP, 128)
    rows = _sc_gather(table, idx3)
    embf = rows.reshape(F, B, 128)
    return _tc_mlp(
        xnum, embf, W0[:NUM], W0[NUM:],
        b0.reshape(1, H), g0.reshape(1, H), be0.reshape(1, H),
        W1, b1.reshape(1, H), g1.reshape(1, H), be1.reshape(1, H),
        W2, b2.reshape(1, H), g2.reshape(1, H), be2.reshape(1, H),
        Wout, bout.reshape(1, C))


# speedup vs baseline: 1.9084x; 1.0059x over previous
import functools

import jax
import jax.numpy as jnp
from jax import lax
from jax.experimental import pallas as pl
from jax.experimental.pallas import tpu as pltpu
from jax.experimental.pallas import tpu_sc as plsc

B = 16384
NUM = 13
F = 26
V = 100000
D = 32
H = 256
C = 7

VP = 100352
TV = VP // 4
NJ = 4

_NC, _NS = 2, 16
_NW = _NC * _NS
_RPW = B * F // _NW
_CHUNK = 104
_NCHUNK = _RPW // _CHUNK
_KF = 8
_NOUT = _NCHUNK // _KF

BLK = 512
NB = B // BLK
TBLK = 2048


def _pad_body(in_ref, e_ref, out_ref):
    x = in_ref[0]
    out_ref[0] = lax.dot_general(
        x, e_ref[...], (((0,), (0,)), ((), ())),
        preferred_element_type=jnp.float32)


def _tc_pad(embT):
    eye = jnp.eye(D, 128, dtype=jnp.float32)
    return pl.pallas_call(
        _pad_body,
        grid=(F, NJ),
        in_specs=[
            pl.BlockSpec((1, D, TV), lambda f, j: (f, 0, j)),
            pl.BlockSpec((D, 128), lambda f, j: (0, 0)),
        ],
        out_specs=pl.BlockSpec((1, TV, 128), lambda f, j: (f, j, 0)),
        out_shape=jax.ShapeDtypeStruct((F, VP, 128), jnp.float32),
    )(embT, eye)


def _sc_gather(table, idx3):
    mesh = plsc.VectorSubcoreMesh(core_axis_name="c", subcore_axis_name="s")

    @functools.partial(
        pl.kernel,
        mesh=mesh,
        out_type=jax.ShapeDtypeStruct((F * B, 128), jnp.float32),
        scratch_types=[
            pltpu.VMEM((_NCHUNK, _CHUNK), jnp.int32),
            pltpu.VMEM((_KF * _CHUNK, 128), jnp.float32),
            pltpu.SemaphoreType.DMA,
        ],
        compiler_params=pltpu.CompilerParams(use_tc_tiling_on_sc=True),
    )
    def gk(table_hbm, idx_hbm, out_hbm, idx_v, rows_v, sem):
        wid = lax.axis_index("s") * _NC + lax.axis_index("c")
        base = wid * _RPW
        pltpu.sync_copy(idx_hbm.at[wid], idx_v)

        def outer(t, carry):
            cps = []
            for k in range(_KF):
                cps.append(pltpu.async_copy(
                    table_hbm.at[idx_v.at[t * _KF + k]],
                    rows_v.at[pl.ds(k * _CHUNK, _CHUNK)],
                    sem))
            for cp in cps:
                cp.wait()
            pltpu.sync_copy(
                rows_v,
                out_hbm.at[pl.ds(base + t * (_KF * _CHUNK), _KF * _CHUNK)])
            return carry

        lax.fori_loop(0, _NOUT, outer, 0)

    return gk(table, idx3)


def _mlp_body(xnum_ref, embf_ref, w0n_ref, w0e_ref, b0_ref, g0_ref, be0_ref,
              w1_ref, b1_ref, g1_ref, be1_ref,
              w2_ref, b2_ref, g2_ref, be2_ref,
              wout_ref, bout_ref, out_ref, zbuf, acc, ebuf):
    t = pl.program_id(0)

    @pl.when(t < NB)
    def _layer0():
        @pl.when(t == 0)
        def _init():
            acc[...] = jnp.zeros_like(acc[...])

        for f in range(F):
            ebuf[:, f * D:(f + 1) * D] = embf_ref[f, :, :D]
        z = (jnp.dot(xnum_ref[...], w0n_ref[...],
                     preferred_element_type=jnp.float32)
             + jnp.dot(ebuf[...], w0e_ref[...],
                       preferred_element_type=jnp.float32)
             + b0_ref[...])
        zbuf[pl.ds(t * BLK, BLK), :] = z
        acc[0:1, :] += jnp.sum(z, axis=0, keepdims=True)
        acc[1:2, :] += jnp.sum(z * z, axis=0, keepdims=True)

    def _affine(g_ref, be_ref):
        mean = acc[0:1, :] * (1.0 / B)
        var = acc[1:2, :] * (1.0 / B) - mean * mean
        scale = g_ref[...] * lax.rsqrt(var + 1e-5)
        shift = be_ref[...] - mean * scale
        return scale, shift

    def _mid(w_ref, b_ref, g_ref, be_ref):
        scale, shift = _affine(g_ref, be_ref)

        def body(j, s):
            zs = zbuf[pl.ds(j * TBLK, TBLK), :]
            h = jnp.maximum(zs * scale + shift, 0.0)
            z = jnp.dot(h, w_ref[...],
                        preferred_element_type=jnp.float32) + b_ref[...]
            zbuf[pl.ds(j * TBLK, TBLK), :] = z
            return (s[0] + jnp.sum(z, axis=0, keepdims=True),
                    s[1] + jnp.sum(z * z, axis=0, keepdims=True))

        s0 = (jnp.zeros((1, H), jnp.float32), jnp.zeros((1, H), jnp.float32))
        s = lax.fori_loop(0, B // TBLK, body, s0)
        acc[0:1, :] = s[0]
        acc[1:2, :] = s[1]

    @pl.when(t == NB)
    def _layer1():
        _mid(w1_ref, b1_ref, g0_ref, be0_ref)

    @pl.when(t == NB + 1)
    def _layer2():
        _mid(w2_ref, b2_ref, g1_ref, be1_ref)

    @pl.when(t == NB + 2)
    def _head():
        scale, shift = _affine(g2_ref, be2_ref)

        def body(j, carry):
            zs = zbuf[pl.ds(j * TBLK, TBLK), :]
            h = jnp.maximum(zs * scale + shift, 0.0)
            out_ref[pl.ds(j * TBLK, TBLK), :] = (
                jnp.dot(h, wout_ref[...],
                        preferred_element_type=jnp.float32) + bout_ref[...])
            return carry

        lax.fori_loop(0, B // TBLK, body, 0)


def _tc_mlp(xnum, embf, w0n, w0e, b0, g0, be0,
            W1, b1, g1, be1, W2, b2, g2, be2, Wout, bout):
    def full(shape):
        return pl.BlockSpec(shape, lambda t: (0, 0))

    def inb(t):
        return (jnp.minimum(t, NB - 1), 0)

    return pl.pallas_call(
        _mlp_body,
        grid=(NB + 3,),
        in_specs=[
            pl.BlockSpec((BLK, NUM), inb),
            pl.BlockSpec((F, BLK, 128),
                         lambda t: (0, jnp.minimum(t, NB - 1), 0)),
            full((NUM, H)), full((F * D, H)),
            full((1, H)), full((1, H)), full((1, H)),
            full((H, H)), full((1, H)), full((1, H)), full((1, H)),
            full((H, H)), full((1, H)), full((1, H)), full((1, H)),
            full((H, C)), full((1, C)),
        ],
        out_specs=pl.BlockSpec((B, C), lambda t: (0, 0)),
        out_shape=jax.ShapeDtypeStruct((B, C), jnp.float32),
        scratch_shapes=[
            pltpu.VMEM((B, H), jnp.float32),
            pltpu.VMEM((2, H), jnp.float32),
            pltpu.VMEM((BLK, F * D), jnp.float32),
        ],
        compiler_params=pltpu.CompilerParams(
            dimension_semantics=("arbitrary",)),
    )(xnum, embf, w0n, w0e, b0, g0, be0,
      W1, b1, g1, be1, W2, b2, g2, be2, Wout, bout)


def kernel(x, emb, W0, b0, g0, be0, W1, b1, g1, be1, W2, b2, g2, be2,
           Wout, bout):
    xnum = x[:, :NUM]
    xcat = x[:, NUM:].astype(jnp.int32)
    idx_fm = xcat.T + (jnp.arange(F, dtype=jnp.int32) * VP)[:, None]
    idx3 = idx_fm.reshape(_NW, _NCHUNK, _CHUNK)
    table = _tc_pad(emb.transpose(0, 2, 1)).reshape(F * VP, 128)
    rows = _sc_gather(table, idx3)
    embf = rows.reshape(F, B, 128)
    return _tc_mlp(
        xnum, embf, W0[:NUM], W0[NUM:],
        b0.reshape(1, H), g0.reshape(1, H), be0.reshape(1, H),
        W1, b1.reshape(1, H), g1.reshape(1, H), be1.reshape(1, H),
        W2, b2.reshape(1, H), g2.reshape(1, H), be2.reshape(1, H),
        Wout, bout.reshape(1, C))
